# Initial kernel scaffold; baseline (speedup 1.0000x reference)
import jax
import jax.numpy as jnp
from jax.experimental import pallas as pl


def kernel(x, table):
    raise NotImplementedError("write your pallas kernel here")


# speedup vs baseline: 3.3327x; 3.3327x over previous
import functools

import jax
import jax.numpy as jnp
from jax import lax
from jax.experimental import pallas as pl
from jax.experimental.pallas import tpu as pltpu
from jax.experimental.pallas import tpu_sc as plsc

NTOKEN = 100000
EMB_DIM = 128
BATCH = 4096
HIST = 50

NC = 2
NS = 16
NW = NC * NS

TOTAL = BATCH * HIST
PER_W = TOTAL // NW
CHUNK = 128
NCHUNK = PER_W // CHUNK


def _make_gather():
  mesh = plsc.VectorSubcoreMesh(core_axis_name="c", subcore_axis_name="s")

  @functools.partial(
      pl.kernel,
      mesh=mesh,
      out_type=jax.ShapeDtypeStruct((NW, NCHUNK, CHUNK, EMB_DIM),
                                    jnp.float32),
      scratch_types=[
          pltpu.VMEM((NCHUNK, CHUNK), jnp.int32),
          pltpu.VMEM((2, CHUNK, EMB_DIM), jnp.float32),
          pltpu.SemaphoreType.DMA,
          pltpu.SemaphoreType.DMA,
      ],
  )
  def gather_kernel(table_hbm, idx_hbm, out_hbm, idx_v, rows_v, gsem, ssem):
    wid = lax.axis_index("s") * NC + lax.axis_index("c")
    pltpu.sync_copy(idx_hbm.at[wid], idx_v)

    pltpu.async_copy(table_hbm.at[idx_v.at[0]], rows_v.at[0], gsem)

    def step(j, carry):
      buf = lax.rem(j, 2)
      nbuf = 1 - buf
      @pl.when(j > 0)
      def _():
        pltpu.make_async_copy(rows_v.at[nbuf], out_hbm.at[wid, j - 1],
                              ssem).wait()
      @pl.when(j + 1 < NCHUNK)
      def _():
        pltpu.async_copy(table_hbm.at[idx_v.at[j + 1]], rows_v.at[nbuf],
                         gsem)
      pltpu.make_async_copy(table_hbm.at[idx_v.at[j]], rows_v.at[buf],
                            gsem).wait()
      pltpu.make_async_copy(rows_v.at[buf], out_hbm.at[wid, j], ssem).start()
      return carry

    lax.fori_loop(0, NCHUNK, step, 0)
    last = NCHUNK - 1
    pltpu.make_async_copy(rows_v.at[last % 2], out_hbm.at[wid, last],
                          ssem).wait()

  return gather_kernel


_gather = _make_gather()


def kernel(x, table):
  idx = x.reshape(NW, NCHUNK, CHUNK).astype(jnp.int32)
  out = _gather(table, idx)
  return out.reshape(BATCH, HIST, EMB_DIM)


# speedup vs baseline: 3.3609x; 1.0085x over previous
import functools

import jax
import jax.numpy as jnp
from jax import lax
from jax.experimental import pallas as pl
from jax.experimental.pallas import tpu as pltpu
from jax.experimental.pallas import tpu_sc as plsc

NTOKEN = 100000
EMB_DIM = 128
BATCH = 4096
HIST = 50

NC = 2
NS = 16
NW = NC * NS

TOTAL = BATCH * HIST
PER_W = TOTAL // NW
CHUNK = 128
NCHUNK = PER_W // CHUNK
NBUF = 6
GAHEAD = 3


def _make_gather():
  mesh = plsc.VectorSubcoreMesh(core_axis_name="c", subcore_axis_name="s")

  @functools.partial(
      pl.kernel,
      mesh=mesh,
      out_type=jax.ShapeDtypeStruct((NW, NCHUNK, CHUNK, EMB_DIM),
                                    jnp.float32),
      scratch_types=[
          pltpu.VMEM((NCHUNK, CHUNK), jnp.int32),
          pltpu.VMEM((NBUF, CHUNK, EMB_DIM), jnp.float32),
          pltpu.SemaphoreType.DMA,
          pltpu.SemaphoreType.DMA,
      ],
  )
  def gather_kernel(table_hbm, idx_hbm, out_hbm, idx_v, rows_v, gsem, ssem):
    wid = lax.axis_index("s") * NC + lax.axis_index("c")
    pltpu.sync_copy(idx_hbm.at[wid], idx_v)

    for j in range(GAHEAD):
      pltpu.async_copy(table_hbm.at[idx_v.at[j]], rows_v.at[j], gsem)

    def step(j, carry):
      buf = lax.rem(j, NBUF)
      @pl.when(j >= NBUF - GAHEAD)
      def _():
        jd = j - (NBUF - GAHEAD)
        pltpu.make_async_copy(rows_v.at[lax.rem(jd, NBUF)],
                              out_hbm.at[wid, jd], ssem).wait()
      @pl.when(j + GAHEAD < NCHUNK)
      def _():
        pltpu.async_copy(table_hbm.at[idx_v.at[j + GAHEAD]],
                         rows_v.at[lax.rem(j + GAHEAD, NBUF)], gsem)
      pltpu.make_async_copy(table_hbm.at[idx_v.at[j]], rows_v.at[buf],
                            gsem).wait()
      pltpu.make_async_copy(rows_v.at[buf], out_hbm.at[wid, j], ssem).start()
      return carry

    lax.fori_loop(0, NCHUNK, step, 0)
    for j in range(NCHUNK - (NBUF - GAHEAD), NCHUNK):
      pltpu.make_async_copy(rows_v.at[j % NBUF], out_hbm.at[wid, j],
                            ssem).wait()

  return gather_kernel


_gather = _make_gather()


def kernel(x, table):
  idx = x.reshape(NW, NCHUNK, CHUNK).astype(jnp.int32)
  out = _gather(table, idx)
  return out.reshape(BATCH, HIST, EMB_DIM)


# speedup vs baseline: 6.0087x; 1.7878x over previous
import functools

import jax
import jax.numpy as jnp
from jax import lax
from jax.experimental import pallas as pl
from jax.experimental.pallas import tpu as pltpu
from jax.experimental.pallas import tpu_sc as plsc

NTOKEN = 100000
EMB_DIM = 128
BATCH = 4096
HIST = 50

NC = 2
NS = 16
NW = NC * NS

ROWS_W = BATCH // NW
NBUF = 8
GAHEAD = 4


def _make_gather():
  mesh = plsc.VectorSubcoreMesh(core_axis_name="c", subcore_axis_name="s")

  @functools.partial(
      pl.kernel,
      mesh=mesh,
      out_type=jax.ShapeDtypeStruct((BATCH, HIST, EMB_DIM), jnp.float32),
      scratch_types=[
          pltpu.VMEM((ROWS_W, HIST), jnp.int32),
          pltpu.VMEM((NBUF, HIST, EMB_DIM), jnp.float32),
          pltpu.SemaphoreType.DMA,
          pltpu.SemaphoreType.DMA,
      ],
      compiler_params=pltpu.CompilerParams(use_tc_tiling_on_sc=True),
  )
  def gather_kernel(table_hbm, idx_hbm, out_hbm, idx_v, rows_v, gsem, ssem):
    wid = lax.axis_index("s") * NC + lax.axis_index("c")
    base = wid * ROWS_W
    pltpu.sync_copy(idx_hbm.at[pl.ds(base, ROWS_W)], idx_v)

    for i in range(GAHEAD):
      pltpu.async_copy(table_hbm.at[idx_v.at[i]], rows_v.at[i], gsem)

    def step(i, carry):
      buf = lax.rem(i, NBUF)
      @pl.when(i >= NBUF - GAHEAD)
      def _():
        jd = i - (NBUF - GAHEAD)
        pltpu.make_async_copy(rows_v.at[lax.rem(jd, NBUF)],
                              out_hbm.at[base + jd], ssem).wait()
      @pl.when(i + GAHEAD < ROWS_W)
      def _():
        pltpu.async_copy(table_hbm.at[idx_v.at[i + GAHEAD]],
                         rows_v.at[lax.rem(i + GAHEAD, NBUF)], gsem)
      pltpu.make_async_copy(table_hbm.at[idx_v.at[i]], rows_v.at[buf],
                            gsem).wait()
      pltpu.make_async_copy(rows_v.at[buf], out_hbm.at[base + i],
                            ssem).start()
      return carry

    lax.fori_loop(0, ROWS_W, step, 0)
    for i in range(ROWS_W - (NBUF - GAHEAD), ROWS_W):
      pltpu.make_async_copy(rows_v.at[i % NBUF], out_hbm.at[base + i],
                            ssem).wait()

  return gather_kernel


_gather = _make_gather()


def kernel(x, table):
  return _gather(table, x.astype(jnp.int32))


# speedup vs baseline: 10.6719x; 1.7761x over previous
import functools

import jax
import jax.numpy as jnp
from jax import lax
from jax.experimental import pallas as pl
from jax.experimental.pallas import tpu as pltpu
from jax.experimental.pallas import tpu_sc as plsc

NTOKEN = 100000
EMB_DIM = 128
BATCH = 4096
HIST = 50

NC = 2
NS = 16
NW = NC * NS

TOTAL = BATCH * HIST
PER_W = TOTAL // NW
CHUNK = 128
NCHUNK = PER_W // CHUNK
NBUF = 7
GAHEAD = 4


def _make_gather():
  mesh = plsc.VectorSubcoreMesh(core_axis_name="c", subcore_axis_name="s")

  @functools.partial(
      pl.kernel,
      mesh=mesh,
      out_type=jax.ShapeDtypeStruct((NW, NCHUNK, CHUNK, EMB_DIM),
                                    jnp.float32),
      scratch_types=[
          pltpu.VMEM((NCHUNK, CHUNK), jnp.int32),
          pltpu.VMEM((NBUF, CHUNK, EMB_DIM), jnp.float32),
          pltpu.SemaphoreType.DMA,
          pltpu.SemaphoreType.DMA,
      ],
      compiler_params=pltpu.CompilerParams(use_tc_tiling_on_sc=True),
  )
  def gather_kernel(table_hbm, idx_hbm, out_hbm, idx_v, rows_v, gsem, ssem):
    wid = lax.axis_index("s") * NC + lax.axis_index("c")
    pltpu.sync_copy(idx_hbm.at[wid], idx_v)

    for j in range(GAHEAD):
      pltpu.async_copy(table_hbm.at[idx_v.at[j]], rows_v.at[j], gsem)

    def step(j, carry):
      buf = lax.rem(j, NBUF)
      @pl.when(j >= NBUF - GAHEAD)
      def _():
        jd = j - (NBUF - GAHEAD)
        pltpu.make_async_copy(rows_v.at[lax.rem(jd, NBUF)],
                              out_hbm.at[wid, jd], ssem).wait()
      @pl.when(j + GAHEAD < NCHUNK)
      def _():
        pltpu.async_copy(table_hbm.at[idx_v.at[j + GAHEAD]],
                         rows_v.at[lax.rem(j + GAHEAD, NBUF)], gsem)
      pltpu.make_async_copy(table_hbm.at[idx_v.at[j]], rows_v.at[buf],
                            gsem).wait()
      pltpu.make_async_copy(rows_v.at[buf], out_hbm.at[wid, j], ssem).start()
      return carry

    lax.fori_loop(0, NCHUNK, step, 0)
    for j in range(NCHUNK - (NBUF - GAHEAD), NCHUNK):
      pltpu.make_async_copy(rows_v.at[j % NBUF], out_hbm.at[wid, j],
                            ssem).wait()

  return gather_kernel


_gather = _make_gather()


def kernel(x, table):
  xt = x.T.astype(jnp.int32).reshape(NW, NCHUNK, CHUNK)
  out = _gather(table, xt)
  return out.reshape(HIST, BATCH, EMB_DIM).transpose(1, 0, 2)


# speedup vs baseline: 10.8356x; 1.0153x over previous
import functools

import jax
import jax.numpy as jnp
from jax import lax
from jax.experimental import pallas as pl
from jax.experimental.pallas import tpu as pltpu
from jax.experimental.pallas import tpu_sc as plsc

NTOKEN = 100000
EMB_DIM = 128
BATCH = 4096
HIST = 50

NC = 2
NS = 16
NW = NC * NS

BLOCK = BATCH // NW
NCHUNK = HIST
NBUF = 7
GAHEAD = 3


def _make_gather():
  mesh = plsc.VectorSubcoreMesh(core_axis_name="c", subcore_axis_name="s")

  @functools.partial(
      pl.kernel,
      mesh=mesh,
      out_type=jax.ShapeDtypeStruct((HIST, BATCH, EMB_DIM), jnp.float32),
      scratch_types=[
          pltpu.VMEM((NCHUNK, BLOCK), jnp.int32),
          pltpu.VMEM((NBUF, BLOCK, EMB_DIM), jnp.float32),
          pltpu.SemaphoreType.DMA,
          pltpu.SemaphoreType.DMA,
      ],
      compiler_params=pltpu.CompilerParams(use_tc_tiling_on_sc=True),
  )
  def gather_kernel(table_hbm, idx_hbm, out_hbm, idx_v, rows_v, gsem, ssem):
    wid = lax.axis_index("s") * NC + lax.axis_index("c")
    col = wid * BLOCK
    pltpu.sync_copy(idx_hbm.at[:, pl.ds(col, BLOCK)], idx_v)

    for j in range(GAHEAD):
      pltpu.async_copy(table_hbm.at[idx_v.at[j]], rows_v.at[j], gsem)

    def step(j, carry):
      buf = lax.rem(j, NBUF)
      @pl.when(j >= NBUF - GAHEAD)
      def _():
        jd = j - (NBUF - GAHEAD)
        pltpu.make_async_copy(rows_v.at[lax.rem(jd, NBUF)],
                              out_hbm.at[jd, pl.ds(col, BLOCK)], ssem).wait()
      @pl.when(j + GAHEAD < NCHUNK)
      def _():
        pltpu.async_copy(table_hbm.at[idx_v.at[j + GAHEAD]],
                         rows_v.at[lax.rem(j + GAHEAD, NBUF)], gsem)
      pltpu.make_async_copy(table_hbm.at[idx_v.at[j]], rows_v.at[buf],
                            gsem).wait()
      pltpu.make_async_copy(rows_v.at[buf], out_hbm.at[j, pl.ds(col, BLOCK)],
                            ssem).start()
      return carry

    lax.fori_loop(0, NCHUNK, step, 0)
    for j in range(NCHUNK - (NBUF - GAHEAD), NCHUNK):
      pltpu.make_async_copy(rows_v.at[j % NBUF],
                            out_hbm.at[j, pl.ds(col, BLOCK)], ssem).wait()

  return gather_kernel


_gather = _make_gather()


def kernel(x, table):
  out = _gather(table, x.T.astype(jnp.int32))
  return out.transpose(1, 0, 2)
